# baseline (speedup 1.0000x reference)
import jax
import jax.numpy as jnp
from jax import lax
from jax.experimental import pallas as pl
from jax.experimental.pallas import tpu as pltpu
from jax.experimental.pallas import tpu_sc as plsc

_NUM_ENTITY = 1000000
_EMBED = 64
_HIDDEN = 128
_OUT = 1
_BATCH = 16384

_TBLK = 32768
_QBLK = _TBLK // 4
_NBLK = pl.cdiv(_NUM_ENTITY, _TBLK)
_ROWS = _NBLK * _QBLK
_ESH = 15

_NC = 2
_NS = 16
_NW = _NC * _NS
_B_PER_W = _BATCH // _NW
_CHUNK = 128
_NCHUNK = _B_PER_W // _CHUNK
_L = 16


def _pack_body(x_ref, o_ref):
    xb = x_ref[...].astype(jnp.bfloat16)
    xt = xb.T
    w = pltpu.bitcast(xt, jnp.int32)
    o_ref[:, 0:_EMBED] = w[0:_QBLK, :]
    o_ref[:, _EMBED:2 * _EMBED] = w[_QBLK:2 * _QBLK, :]


def _tc_pack(embed):
    return pl.pallas_call(
        _pack_body,
        grid=(_NBLK,),
        in_specs=[pl.BlockSpec((_EMBED, _TBLK), lambda i: (0, i))],
        out_specs=pl.BlockSpec((_QBLK, 2 * _EMBED), lambda i: (i, 0)),
        out_shape=jax.ShapeDtypeStruct((_ROWS, 2 * _EMBED), jnp.int32),
        compiler_params=pltpu.CompilerParams(
            fuse_transposed_lhs_in_matmul=True,
        ),
    )(embed.T)


def _gather_body(table_hbm, idx_hbm, out_hbm, idx_v, rows_v, sem):
    wid = lax.axis_index("s") * _NC + lax.axis_index("c")
    base = wid * _B_PER_W
    pltpu.sync_copy(idx_hbm.at[pl.ds(base, _B_PER_W)], idx_v)
    for k in range(_B_PER_W // _L):
        e = idx_v[pl.ds(k * _L, _L)]
        idx_v[pl.ds(k * _L, _L)] = (
            ((e >> _ESH) << (_ESH - 2)) + ((e & (_TBLK // 2 - 1)) >> 1)
        )
    copies = []
    for j in range(_NCHUNK):
        copies.append(
            pltpu.async_copy(
                table_hbm.at[idx_v.at[pl.ds(j * _CHUNK, _CHUNK)]],
                rows_v.at[pl.ds(j * _CHUNK, _CHUNK)],
                sem,
            )
        )
    for c in copies:
        c.wait()
    pltpu.sync_copy(rows_v, out_hbm.at[pl.ds(base, _B_PER_W)])


def _sc_gather(table, batch):
    mesh = plsc.VectorSubcoreMesh(core_axis_name="c", subcore_axis_name="s")
    k = pl.kernel(
        _gather_body,
        mesh=mesh,
        out_type=jax.ShapeDtypeStruct((_BATCH, 2 * _EMBED), jnp.int32),
        scratch_types=[
            pltpu.VMEM((_B_PER_W,), jnp.int32),
            pltpu.VMEM((_B_PER_W, 2 * _EMBED), jnp.int32),
            pltpu.SemaphoreType.DMA,
        ],
    )
    return k(table, batch)


_BLK = 8192


def _mlp_body(x_ref, b_ref, w1_ref, b1_ref, w2_ref, b2_ref, o_ref):
    xw = lax.bitcast_convert_type(x_ref[...], jnp.uint32)
    b = b_ref[...]
    hf = (b >> (_ESH - 1)) & 1
    s = b & 1
    wsel = jnp.where(hf == 1, xw[:, _EMBED:2 * _EMBED], xw[:, 0:_EMBED])
    bits = jnp.where(s == 1, wsel >> 16, wsel & 0xFFFF)
    xv = lax.bitcast_convert_type(
        bits.astype(jnp.uint16), jnp.bfloat16
    ).astype(jnp.float32)
    h = jnp.dot(xv, w1_ref[...], preferred_element_type=jnp.float32)
    h = jnp.maximum(h + b1_ref[...], 0.0)
    o_ref[...] = (
        jnp.sum(h * w2_ref[...], axis=1, keepdims=True) + b2_ref[...]
    )


def _tc_mlp(x, batch, W1, b1, W2, b2):
    b2d = batch.reshape(_BATCH, 1)
    b1r = b1.reshape(1, _HIDDEN)
    w2r = W2.reshape(1, _HIDDEN)
    b2r = b2.reshape(1, 1)
    return pl.pallas_call(
        _mlp_body,
        grid=(_BATCH // _BLK,),
        in_specs=[
            pl.BlockSpec((_BLK, 2 * _EMBED), lambda i: (i, 0)),
            pl.BlockSpec((_BLK, 1), lambda i: (i, 0)),
            pl.BlockSpec((_EMBED, _HIDDEN), lambda i: (0, 0)),
            pl.BlockSpec((1, _HIDDEN), lambda i: (0, 0)),
            pl.BlockSpec((1, _HIDDEN), lambda i: (0, 0)),
            pl.BlockSpec((1, 1), lambda i: (0, 0)),
        ],
        out_specs=pl.BlockSpec((_BLK, _OUT), lambda i: (i, 0)),
        out_shape=jax.ShapeDtypeStruct((_BATCH, _OUT), jnp.float32),
    )(x, b2d, W1, b1r, w2r, b2r)


def kernel(batch, embed, W1, b1, W2, b2):
    table = _tc_pack(embed)
    x = _sc_gather(table, batch)
    return _tc_mlp(x, batch, W1, b1, W2, b2)


# speedup vs baseline: 1.0120x; 1.0120x over previous
import jax
import jax.numpy as jnp
from jax import lax
from jax.experimental import pallas as pl
from jax.experimental.pallas import tpu as pltpu
from jax.experimental.pallas import tpu_sc as plsc

_NUM_ENTITY = 1000000
_EMBED = 64
_HIDDEN = 128
_OUT = 1
_BATCH = 16384

_TBLK = 32768
_QBLK = _TBLK // 4
_NBLK = pl.cdiv(_NUM_ENTITY, _TBLK)
_ROWS = _NBLK * _QBLK
_ESH = 15

_NC = 2
_NS = 16
_NW = _NC * _NS
_B_PER_W = _BATCH // _NW
_CHUNK = 128
_NCHUNK = _B_PER_W // _CHUNK
_L = 16


def _pack_body(x_ref, o_ref):
    xb = x_ref[...].astype(jnp.bfloat16)
    xt = xb.T
    w = pltpu.bitcast(xt, jnp.int32)
    o_ref[:, 0:_EMBED] = w[0:_QBLK, :]
    o_ref[:, _EMBED:2 * _EMBED] = w[_QBLK:2 * _QBLK, :]


def _tc_pack(embed):
    return pl.pallas_call(
        _pack_body,
        grid=(_NBLK,),
        in_specs=[pl.BlockSpec((_EMBED, _TBLK), lambda i: (0, i))],
        out_specs=pl.BlockSpec((_QBLK, 2 * _EMBED), lambda i: (i, 0)),
        out_shape=jax.ShapeDtypeStruct((_ROWS, 2 * _EMBED), jnp.int32),
        compiler_params=pltpu.CompilerParams(
            fuse_transposed_lhs_in_matmul=True,
        ),
    )(embed.T)


def _gather_body(table_hbm, idx_hbm, out_hbm, idx_v, rows_v, sem):
    wid = lax.axis_index("s") * _NC + lax.axis_index("c")
    base = wid * _B_PER_W
    pltpu.sync_copy(idx_hbm.at[pl.ds(base, _B_PER_W)], idx_v)
    for k in range(_B_PER_W // _L):
        e = idx_v[pl.ds(k * _L, _L)]
        idx_v[pl.ds(k * _L, _L)] = (
            ((e >> _ESH) << (_ESH - 2)) + ((e & (_TBLK // 2 - 1)) >> 1)
        )
    copies = []
    for j in range(_NCHUNK):
        copies.append(
            pltpu.async_copy(
                table_hbm.at[idx_v.at[pl.ds(j * _CHUNK, _CHUNK)]],
                rows_v.at[pl.ds(j * _CHUNK, _CHUNK)],
                sem,
            )
        )
    for c in copies:
        c.wait()
    pltpu.sync_copy(rows_v, out_hbm.at[pl.ds(base, _B_PER_W)])


def _sc_gather(table, batch):
    mesh = plsc.VectorSubcoreMesh(core_axis_name="c", subcore_axis_name="s")
    k = pl.kernel(
        _gather_body,
        mesh=mesh,
        out_type=jax.ShapeDtypeStruct((_BATCH, 2 * _EMBED), jnp.int32),
        scratch_types=[
            pltpu.VMEM((_B_PER_W,), jnp.int32),
            pltpu.VMEM((_B_PER_W, 2 * _EMBED), jnp.int32),
            pltpu.SemaphoreType.DMA,
        ],
    )
    return k(table, batch)


_BLK = 4096


def _mlp_body(x_ref, b_ref, w1_ref, b1_ref, w2_ref, b2_ref, o_ref):
    xw = lax.bitcast_convert_type(x_ref[...], jnp.uint32)
    b = b_ref[...]
    hf = (b >> (_ESH - 1)) & 1
    s = b & 1
    wsel = jnp.where(hf == 1, xw[:, _EMBED:2 * _EMBED], xw[:, 0:_EMBED])
    bits = jnp.where(s == 1, wsel >> 16, wsel & 0xFFFF)
    xv = lax.bitcast_convert_type(
        bits.astype(jnp.uint16), jnp.bfloat16
    ).astype(jnp.float32)
    h = jnp.dot(xv, w1_ref[...], preferred_element_type=jnp.float32)
    h = jnp.maximum(h + b1_ref[...], 0.0)
    o_ref[...] = (
        jnp.sum(h * w2_ref[...], axis=1, keepdims=True) + b2_ref[...]
    )


def _tc_mlp(x, batch, W1, b1, W2, b2):
    b2d = batch.reshape(_BATCH, 1)
    b1r = b1.reshape(1, _HIDDEN)
    w2r = W2.reshape(1, _HIDDEN)
    b2r = b2.reshape(1, 1)
    return pl.pallas_call(
        _mlp_body,
        grid=(_BATCH // _BLK,),
        in_specs=[
            pl.BlockSpec((_BLK, 2 * _EMBED), lambda i: (i, 0)),
            pl.BlockSpec((_BLK, 1), lambda i: (i, 0)),
            pl.BlockSpec((_EMBED, _HIDDEN), lambda i: (0, 0)),
            pl.BlockSpec((1, _HIDDEN), lambda i: (0, 0)),
            pl.BlockSpec((1, _HIDDEN), lambda i: (0, 0)),
            pl.BlockSpec((1, 1), lambda i: (0, 0)),
        ],
        out_specs=pl.BlockSpec((_BLK, _OUT), lambda i: (i, 0)),
        out_shape=jax.ShapeDtypeStruct((_BATCH, _OUT), jnp.float32),
    )(x, b2d, W1, b1r, w2r, b2r)


def kernel(batch, embed, W1, b1, W2, b2):
    table = _tc_pack(embed)
    x = _sc_gather(table, batch)
    return _tc_mlp(x, batch, W1, b1, W2, b2)


# speedup vs baseline: 1.0199x; 1.0078x over previous
import jax
import jax.numpy as jnp
from jax import lax
from jax.experimental import pallas as pl
from jax.experimental.pallas import tpu as pltpu
from jax.experimental.pallas import tpu_sc as plsc

_NUM_ENTITY = 1000000
_EMBED = 64
_HIDDEN = 128
_OUT = 1
_BATCH = 16384

_TBLK = 32768
_QBLK = _TBLK // 4
_NBLK = pl.cdiv(_NUM_ENTITY, _TBLK)
_ROWS = _NBLK * _QBLK
_ESH = 15

_NC = 2
_NS = 16
_NW = _NC * _NS
_B_PER_W = _BATCH // _NW
_CHUNK = 128
_NCHUNK = _B_PER_W // _CHUNK
_L = 16


def _pack_body(x_ref, o_ref):
    xb = x_ref[...].astype(jnp.bfloat16)
    xt = xb.T
    w = pltpu.bitcast(xt, jnp.int32)
    o_ref[:, 0:_EMBED] = w[0:_QBLK, :]
    o_ref[:, _EMBED:2 * _EMBED] = w[_QBLK:2 * _QBLK, :]


def _tc_pack(embed):
    return pl.pallas_call(
        _pack_body,
        grid=(_NBLK,),
        in_specs=[pl.BlockSpec((_EMBED, _TBLK), lambda i: (0, i))],
        out_specs=pl.BlockSpec((_QBLK, 2 * _EMBED), lambda i: (i, 0)),
        out_shape=jax.ShapeDtypeStruct((_ROWS, 2 * _EMBED), jnp.int32),
        compiler_params=pltpu.CompilerParams(
            vmem_limit_bytes=100_000_000,
        ),
    )(embed.T)


def _gather_body(table_hbm, idx_hbm, out_hbm, idx_v, rows_v, sem):
    wid = lax.axis_index("s") * _NC + lax.axis_index("c")
    base = wid * _B_PER_W
    pltpu.sync_copy(idx_hbm.at[pl.ds(base, _B_PER_W)], idx_v)
    for k in range(_B_PER_W // _L):
        e = idx_v[pl.ds(k * _L, _L)]
        idx_v[pl.ds(k * _L, _L)] = (
            ((e >> _ESH) << (_ESH - 2)) + ((e & (_TBLK // 2 - 1)) >> 1)
        )
    copies = []
    for j in range(_NCHUNK):
        copies.append(
            pltpu.async_copy(
                table_hbm.at[idx_v.at[pl.ds(j * _CHUNK, _CHUNK)]],
                rows_v.at[pl.ds(j * _CHUNK, _CHUNK)],
                sem,
            )
        )
    for c in copies:
        c.wait()
    pltpu.sync_copy(rows_v, out_hbm.at[pl.ds(base, _B_PER_W)])


def _sc_gather(table, batch):
    mesh = plsc.VectorSubcoreMesh(core_axis_name="c", subcore_axis_name="s")
    k = pl.kernel(
        _gather_body,
        mesh=mesh,
        out_type=jax.ShapeDtypeStruct((_BATCH, 2 * _EMBED), jnp.int32),
        scratch_types=[
            pltpu.VMEM((_B_PER_W,), jnp.int32),
            pltpu.VMEM((_B_PER_W, 2 * _EMBED), jnp.int32),
            pltpu.SemaphoreType.DMA,
        ],
    )
    return k(table, batch)


_BLK = 4096


def _mlp_body(x_ref, b_ref, w1_ref, b1_ref, w2_ref, b2_ref, o_ref):
    xw = lax.bitcast_convert_type(x_ref[...], jnp.uint32)
    b = b_ref[...]
    hf = (b >> (_ESH - 1)) & 1
    s = b & 1
    wsel = jnp.where(hf == 1, xw[:, _EMBED:2 * _EMBED], xw[:, 0:_EMBED])
    bits = jnp.where(s == 1, wsel >> 16, wsel & 0xFFFF)
    xv = lax.bitcast_convert_type(
        bits.astype(jnp.uint16), jnp.bfloat16
    ).astype(jnp.float32)
    h = jnp.dot(xv, w1_ref[...], preferred_element_type=jnp.float32)
    h = jnp.maximum(h + b1_ref[...], 0.0)
    o_ref[...] = (
        jnp.sum(h * w2_ref[...], axis=1, keepdims=True) + b2_ref[...]
    )


def _tc_mlp(x, batch, W1, b1, W2, b2):
    b2d = batch.reshape(_BATCH, 1)
    b1r = b1.reshape(1, _HIDDEN)
    w2r = W2.reshape(1, _HIDDEN)
    b2r = b2.reshape(1, 1)
    return pl.pallas_call(
        _mlp_body,
        grid=(_BATCH // _BLK,),
        in_specs=[
            pl.BlockSpec((_BLK, 2 * _EMBED), lambda i: (i, 0)),
            pl.BlockSpec((_BLK, 1), lambda i: (i, 0)),
            pl.BlockSpec((_EMBED, _HIDDEN), lambda i: (0, 0)),
            pl.BlockSpec((1, _HIDDEN), lambda i: (0, 0)),
            pl.BlockSpec((1, _HIDDEN), lambda i: (0, 0)),
            pl.BlockSpec((1, 1), lambda i: (0, 0)),
        ],
        out_specs=pl.BlockSpec((_BLK, _OUT), lambda i: (i, 0)),
        out_shape=jax.ShapeDtypeStruct((_BATCH, _OUT), jnp.float32),
    )(x, b2d, W1, b1r, w2r, b2r)


def kernel(batch, embed, W1, b1, W2, b2):
    table = _tc_pack(embed)
    x = _sc_gather(table, batch)
    return _tc_mlp(x, batch, W1, b1, W2, b2)
